# baseline (speedup 1.0000x reference)
import functools

import jax
import jax.numpy as jnp
from jax import lax
from jax.experimental import pallas as pl
from jax.experimental.pallas import tpu as pltpu
from jax.experimental.pallas import tpu_sc as plsc

_MAX_NORM = 1.0
_EPS = 1e-7
_CHUNK = 128


def _rsqrt_nr(x):
    i = plsc.bitcast(x, jnp.int32)
    i = jnp.int32(0x5F3759DF) - (i >> 1)
    y = plsc.bitcast(i, jnp.float32)
    for _ in range(3):
        y = y * (1.5 - 0.5 * x * y * y)
    return y


@functools.cache
def _build(NW, NC, NCH, C, D, B):
    bpw = NCH * C
    mesh = plsc.VectorSubcoreMesh(core_axis_name="c", subcore_axis_name="s")

    @functools.partial(
        pl.kernel,
        mesh=mesh,
        out_type=jax.ShapeDtypeStruct((B,), jnp.float32),
        scratch_types=[
            pltpu.VMEM((NCH, C), jnp.int32),
            pltpu.VMEM((NCH, C), jnp.int32),
            pltpu.VMEM((bpw, D), jnp.float32),
            pltpu.VMEM((bpw, D), jnp.float32),
            pltpu.VMEM((bpw,), jnp.float32),
            pltpu.SemaphoreType.DMA,
        ],
        compiler_params=pltpu.CompilerParams(
            needs_layout_passes=False, use_tc_tiling_on_sc=False),
    )
    def k(uids_hbm, iids_hbm, users_hbm, items_hbm, out_hbm,
          uidx, iidx, urows, irows, obuf, sem):
        wid = lax.axis_index("s") * NC + lax.axis_index("c")
        pltpu.sync_copy(uids_hbm.at[wid], uidx)
        pltpu.sync_copy(iids_hbm.at[wid], iidx)
        copies = []
        for j in range(NCH):
            copies.append(pltpu.async_copy(
                users_hbm.at[uidx.at[j]], urows.at[pl.ds(j * C, C)], sem))
            copies.append(pltpu.async_copy(
                items_hbm.at[iidx.at[j]], irows.at[pl.ds(j * C, C)], sem))
        for cp in copies:
            cp.wait()

        lanes = lax.iota(jnp.int32, 16)
        zeros = jnp.zeros((16,), jnp.float32)
        eps2 = jnp.float32(_EPS * _EPS)
        def group_body(g, _):
            rows = g * 16 + lanes

            def d_body(d, carry):
                acc, nu, nv = carry
                cols = jnp.full((16,), d, dtype=jnp.int32)
                u = plsc.load_gather(urows, [rows, cols])
                v = plsc.load_gather(irows, [rows, cols])
                return acc + u * v, nu + u * u, nv + v * v

            acc, nu, nv = lax.fori_loop(0, D, d_body, (zeros, zeros, zeros))
            su = jnp.minimum(jnp.float32(_MAX_NORM),
                             _rsqrt_nr(jnp.maximum(nu, eps2)))
            sv = jnp.minimum(jnp.float32(_MAX_NORM),
                             _rsqrt_nr(jnp.maximum(nv, eps2)))
            x = acc * su * sv
            obuf[pl.ds(g * 16, 16)] = 1.0 / (1.0 + jnp.exp(-x))
            return 0

        lax.fori_loop(0, bpw // 16, group_body, 0)
        pltpu.sync_copy(obuf, out_hbm.at[pl.ds(wid * bpw, bpw)])

    return k


@jax.jit
def kernel(user_ids, item_ids, users, items):
    B = user_ids.shape[0]
    D = users.shape[1]
    info = plsc.get_sparse_core_info()
    NC, NS = info.num_cores, info.num_subcores
    NW = NC * NS
    NCH = B // (NW * _CHUNK)
    uids = user_ids.astype(jnp.int32).reshape(NW, NCH, _CHUNK)
    iids = item_ids.astype(jnp.int32).reshape(NW, NCH, _CHUNK)
    k = _build(NW, NC, NCH, _CHUNK, D, B)
    return k(uids, iids, users, items)
